# Initial kernel scaffold; baseline (speedup 1.0000x reference)
import jax
import jax.numpy as jnp
from jax.experimental import pallas as pl


def kernel(v, e_hr_to_lr, idxHr_to_idxLr, batch_lr, edge_index_lr, edge_attr_lr, W_enc, b_enc, gamma, beta, W0, b0, W1, b1, W_lr, b_lr):
    raise NotImplementedError("write your pallas kernel here")


# speedup vs baseline: 1.0181x; 1.0181x over previous
import functools

import jax
import jax.numpy as jnp
from jax import lax
from jax.experimental import pallas as pl
from jax.experimental.pallas import tpu as pltpu
from jax.experimental.pallas import tpu_sc as plsc

_N_HR = 100000
_N_LR = 25000
_E_LR = 400000
_D_IN = 128
_FW = 64

_NLR_PAD = 25600
_STRIPE = _NLR_PAD // 16
_CHUNK = 80
_N_CHUNKS = _N_HR // _CHUNK
_N_WORKERS = 32
_ITERS = -(-_N_CHUNKS // _N_WORKERS)

_SELU_ALPHA = 1.6732632423543772
_SELU_SCALE = 1.0507009873554805


def _selu(x):
    return _SELU_SCALE * jnp.where(x > 0, x, _SELU_ALPHA * (jnp.exp(x) - 1.0))


def _h_body(v_ref, e_ref, wenc_ref, benc_ref, gamma_ref, beta_ref,
            w0t_ref, b0_ref, w1t_ref, b1_ref, out_ref):
    e = e_ref[...] * wenc_ref[...] + benc_ref[...]
    h = jnp.concatenate([e, v_ref[...]], axis=1)
    mu = jnp.mean(h, axis=1, keepdims=True)
    d = h - mu
    var = jnp.mean(d * d, axis=1, keepdims=True)
    hn = d * lax.rsqrt(var + 1e-5) * gamma_ref[...] + beta_ref[...]
    x = jnp.dot(hn, w0t_ref[...], preferred_element_type=jnp.float32)
    x = _selu(x + b0_ref[...])
    x = jnp.dot(x, w1t_ref[...], preferred_element_type=jnp.float32)
    out_ref[...] = _selu(x + b1_ref[...])


def _compute_h(v, e_hr, wenc_row, benc_row, gamma_row, beta_row, w0t, b0_row, w1t, b1_row):
    blk = 2000
    grid = _N_HR // blk
    full = lambda a: pl.BlockSpec(a.shape, lambda i: (0, 0))
    return pl.pallas_call(
        _h_body,
        grid=(grid,),
        in_specs=[
            pl.BlockSpec((blk, _D_IN), lambda i: (i, 0)),
            pl.BlockSpec((blk, 1), lambda i: (i, 0)),
            full(wenc_row), full(benc_row), full(gamma_row), full(beta_row),
            full(w0t), full(b0_row), full(w1t), full(b1_row),
        ],
        out_specs=pl.BlockSpec((blk, _FW), lambda i: (i, 0)),
        out_shape=jax.ShapeDtypeStruct((_N_HR, _FW), jnp.float32),
    )(v, e_hr, wenc_row, benc_row, gamma_row, beta_row, w0t, b0_row, w1t, b1_row)


def _sc_scatter_body(h_hbm, idx_hbm, z2d_hbm, zc_hbm, ones_hbm,
                     out_sums, out_cnt,
                     acc_s, acc_c, hbuf, ibuf, obuf):
    c = lax.axis_index("c")
    s = lax.axis_index("s")
    wid = c * 16 + s

    pltpu.sync_copy(z2d_hbm.at[pl.ds(s * _STRIPE, _STRIPE)],
                    acc_s.at[pl.ds(s * _STRIPE, _STRIPE)])
    pltpu.sync_copy(zc_hbm.at[pl.ds(s * _STRIPE, _STRIPE)],
                    acc_c.at[pl.ds(s * _STRIPE, _STRIPE)])

    pltpu.sync_copy(ones_hbm, obuf)

    plsc.subcore_barrier()

    def step(i, carry):
        k = wid + _N_WORKERS * i

        @pl.when(k < _N_CHUNKS)
        def _():
            base = k * _CHUNK
            pltpu.sync_copy(h_hbm.at[pl.ds(base, _CHUNK)], hbuf)
            pltpu.sync_copy(idx_hbm.at[pl.ds(base, _CHUNK)], ibuf)
            pltpu.sync_copy(hbuf, acc_s.at[ibuf], add=True)
            pltpu.sync_copy(obuf, acc_c.at[ibuf], add=True)

        return carry

    lax.fori_loop(0, _ITERS, step, 0)

    plsc.subcore_barrier()

    off = c * _NLR_PAD + s * _STRIPE
    pltpu.sync_copy(acc_s.at[pl.ds(s * _STRIPE, _STRIPE)],
                    out_sums.at[pl.ds(off, _STRIPE)])
    pltpu.sync_copy(acc_c.at[pl.ds(s * _STRIPE, _STRIPE)],
                    out_cnt.at[pl.ds(off, _STRIPE)])


@functools.cache
def _sc_scatter():
    return functools.partial(
        pl.kernel,
        out_type=[
            jax.ShapeDtypeStruct((2 * _NLR_PAD, _FW), jnp.float32),
            jax.ShapeDtypeStruct((2 * _NLR_PAD, 8), jnp.float32),
        ],
        mesh=plsc.VectorSubcoreMesh(core_axis_name="c", subcore_axis_name="s"),
        scratch_types=[
            pltpu.VMEM_SHARED((_NLR_PAD, _FW), jnp.float32),
            pltpu.VMEM_SHARED((_NLR_PAD, 8), jnp.float32),
            pltpu.VMEM((_CHUNK, _FW), jnp.float32),
            pltpu.VMEM((_CHUNK,), jnp.int32),
            pltpu.VMEM((_CHUNK, 8), jnp.float32),
        ],
        compiler_params=pltpu.CompilerParams(use_tc_tiling_on_sc=False),
    )(_sc_scatter_body)


def _comb_body(s_ref, c_ref, o_ref):
    ssum = s_ref[0] + s_ref[1]
    cnt = c_ref[0, :, 0:1] + c_ref[1, :, 0:1]
    o_ref[...] = ssum / jnp.maximum(cnt, 1.0)


def _combine(psums, pcnt):
    blk = 1000
    grid = _N_LR // blk
    return pl.pallas_call(
        _comb_body,
        grid=(grid,),
        in_specs=[
            pl.BlockSpec((2, blk, _FW), lambda i: (0, i, 0)),
            pl.BlockSpec((2, blk, 8), lambda i: (0, i, 0)),
        ],
        out_specs=pl.BlockSpec((blk, _FW), lambda i: (i, 0)),
        out_shape=jax.ShapeDtypeStruct((_N_LR, _FW), jnp.float32),
    )(psums, pcnt)


def _edge_body(a_ref, wt_ref, b_ref, o_ref):
    o_ref[...] = (jnp.dot(a_ref[...], wt_ref[...],
                          preferred_element_type=jnp.float32) + b_ref[...])


def _edge_encode(edge_attr, wlrt, blr_row):
    blk = 8000
    grid = _E_LR // blk
    return pl.pallas_call(
        _edge_body,
        grid=(grid,),
        in_specs=[
            pl.BlockSpec((blk, 3), lambda i: (i, 0)),
            pl.BlockSpec(wlrt.shape, lambda i: (0, 0)),
            pl.BlockSpec(blr_row.shape, lambda i: (0, 0)),
        ],
        out_specs=pl.BlockSpec((blk, _FW), lambda i: (i, 0)),
        out_shape=jax.ShapeDtypeStruct((_E_LR, _FW), jnp.float32),
    )(edge_attr, wlrt, blr_row)


@jax.jit
def kernel(v, e_hr_to_lr, idxHr_to_idxLr, batch_lr, edge_index_lr, edge_attr_lr,
           W_enc, b_enc, gamma, beta, W0, b0, W1, b1, W_lr, b_lr):
    wenc_row = W_enc.reshape(1, _FW)
    benc_row = b_enc.reshape(1, _FW)
    gamma_row = gamma.reshape(1, _FW + _D_IN)
    beta_row = beta.reshape(1, _FW + _D_IN)
    w0t = W0.T
    b0_row = b0.reshape(1, _FW)
    w1t = W1.T
    b1_row = b1.reshape(1, _FW)
    wlrt = W_lr.T
    blr_row = b_lr.reshape(1, _FW)

    h = _compute_h(v, e_hr_to_lr, wenc_row, benc_row, gamma_row, beta_row,
                   w0t, b0_row, w1t, b1_row)

    z2d = jnp.zeros((_NLR_PAD, _FW), jnp.float32)
    zc = jnp.zeros((_NLR_PAD, 8), jnp.float32)
    ones = jnp.ones((_CHUNK, 8), jnp.float32)
    flat_sums, flat_cnt = _sc_scatter()(h, idxHr_to_idxLr, z2d, zc, ones)
    psums = flat_sums.reshape(2, _NLR_PAD, _FW)
    pcnt = flat_cnt.reshape(2, _NLR_PAD, 8)

    v_out = _combine(psums, pcnt)
    e_out = _edge_encode(edge_attr_lr, wlrt, blr_row)
    return (v_out, e_out)


# speedup vs baseline: 2.4192x; 2.3762x over previous
import functools

import jax
import jax.numpy as jnp
from jax import lax
from jax.experimental import pallas as pl
from jax.experimental.pallas import tpu as pltpu
from jax.experimental.pallas import tpu_sc as plsc

_N_HR = 100000
_N_LR = 25000
_E_LR = 400000
_D_IN = 128
_FW = 64

_NLR_PAD = 25600
_STRIPE = _NLR_PAD // 16
_CHUNK = 80
_N_CHUNKS = _N_HR // _CHUNK
_N_WORKERS = 32
_ITERS = -(-_N_CHUNKS // _N_WORKERS)

_SELU_ALPHA = 1.6732632423543772
_SELU_SCALE = 1.0507009873554805


def _selu(x):
    return _SELU_SCALE * jnp.where(x > 0, x, _SELU_ALPHA * (jnp.exp(x) - 1.0))


def _h_body(v_ref, e_ref, m1_ref, m2_ref, r0_ref, r1e_ref, s0_ref, c0_ref,
            cs_ref, w1t_ref, b1_ref, out_ref):
    v = v_ref[...]
    a = e_ref[...]
    p = jnp.dot(v, m1_ref[...], preferred_element_type=jnp.float32)
    tv = p[:, :_FW]
    sum_v = p[:, _FW:_FW + 1]
    vsq = v * v
    q = jnp.dot(vsq, m2_ref[...], preferred_element_type=jnp.float32)
    ssq_v = q[:, 0:1]
    sw = cs_ref[0, 0]
    sb = cs_ref[0, 1]
    sw2 = cs_ref[0, 2]
    swb = cs_ref[0, 3]
    sb2 = cs_ref[0, 4]
    inv_d = 1.0 / (_FW + _D_IN)
    mu = (sum_v + a * sw + sb) * inv_d
    msq = (ssq_v + (a * a) * sw2 + 2.0 * (a * swb) + sb2) * inv_d
    istd = lax.rsqrt((msq - mu * mu) + 1e-5)
    x = istd * (tv + a * r0_ref[...] + r1e_ref[...] - mu * s0_ref[...]) + c0_ref[...]
    x = _selu(x)
    y = jnp.dot(x, w1t_ref[...], preferred_element_type=jnp.float32)
    out_ref[...] = _selu(y + b1_ref[...])


def _compute_h(v, e_hr, m1, m2, r0, r1e, s0, c0, cs, w1t, b1_row):
    blk = 2000
    grid = _N_HR // blk
    full = lambda a: pl.BlockSpec(a.shape, lambda i: (0, 0))
    return pl.pallas_call(
        _h_body,
        grid=(grid,),
        in_specs=[
            pl.BlockSpec((blk, _D_IN), lambda i: (i, 0)),
            pl.BlockSpec((blk, 1), lambda i: (i, 0)),
            full(m1), full(m2), full(r0), full(r1e), full(s0), full(c0),
            full(cs), full(w1t), full(b1_row),
        ],
        out_specs=pl.BlockSpec((blk, _FW), lambda i: (i, 0)),
        out_shape=jax.ShapeDtypeStruct((_N_HR, _FW), jnp.float32),
    )(v, e_hr, m1, m2, r0, r1e, s0, c0, cs, w1t, b1_row)


def _sc_scatter_body(h_hbm, idx_hbm, z2d_hbm, zc_hbm, ones_hbm,
                     out_sums, out_cnt,
                     acc_s, acc_c, hbuf, ibuf, obuf):
    c = lax.axis_index("c")
    s = lax.axis_index("s")
    wid = c * 16 + s

    pltpu.sync_copy(z2d_hbm.at[pl.ds(s * _STRIPE, _STRIPE)],
                    acc_s.at[pl.ds(s * _STRIPE, _STRIPE)])
    pltpu.sync_copy(zc_hbm.at[pl.ds(s * _STRIPE, _STRIPE)],
                    acc_c.at[pl.ds(s * _STRIPE, _STRIPE)])

    pltpu.sync_copy(ones_hbm, obuf)

    plsc.subcore_barrier()

    def step(i, carry):
        k = wid + _N_WORKERS * i

        @pl.when(k < _N_CHUNKS)
        def _():
            base = k * _CHUNK
            pltpu.sync_copy(h_hbm.at[pl.ds(base, _CHUNK)], hbuf)
            pltpu.sync_copy(idx_hbm.at[pl.ds(base, _CHUNK)], ibuf)
            pltpu.sync_copy(hbuf, acc_s.at[ibuf], add=True)
            pltpu.sync_copy(obuf, acc_c.at[ibuf], add=True)

        return carry

    lax.fori_loop(0, _ITERS, step, 0)

    plsc.subcore_barrier()

    off = c * _NLR_PAD + s * _STRIPE
    pltpu.sync_copy(acc_s.at[pl.ds(s * _STRIPE, _STRIPE)],
                    out_sums.at[pl.ds(off, _STRIPE)])
    pltpu.sync_copy(acc_c.at[pl.ds(s * _STRIPE, _STRIPE)],
                    out_cnt.at[pl.ds(off, _STRIPE)])


@functools.cache
def _sc_scatter():
    return functools.partial(
        pl.kernel,
        out_type=[
            jax.ShapeDtypeStruct((2 * _NLR_PAD, _FW), jnp.float32),
            jax.ShapeDtypeStruct((2 * _NLR_PAD, 8), jnp.float32),
        ],
        mesh=plsc.VectorSubcoreMesh(core_axis_name="c", subcore_axis_name="s"),
        scratch_types=[
            pltpu.VMEM_SHARED((_NLR_PAD, _FW), jnp.float32),
            pltpu.VMEM_SHARED((_NLR_PAD, 8), jnp.float32),
            pltpu.VMEM((_CHUNK, _FW), jnp.float32),
            pltpu.VMEM((_CHUNK,), jnp.int32),
            pltpu.VMEM((_CHUNK, 8), jnp.float32),
        ],
        compiler_params=pltpu.CompilerParams(use_tc_tiling_on_sc=False),
    )(_sc_scatter_body)


def _comb_body(s_ref, c_ref, o_ref):
    ssum = s_ref[0] + s_ref[1]
    cnt = c_ref[0, :, 0:1] + c_ref[1, :, 0:1]
    o_ref[...] = ssum / jnp.maximum(cnt, 1.0)


def _combine(psums, pcnt):
    blk = 1000
    grid = _N_LR // blk
    return pl.pallas_call(
        _comb_body,
        grid=(grid,),
        in_specs=[
            pl.BlockSpec((2, blk, _FW), lambda i: (0, i, 0)),
            pl.BlockSpec((2, blk, 8), lambda i: (0, i, 0)),
        ],
        out_specs=pl.BlockSpec((blk, _FW), lambda i: (i, 0)),
        out_shape=jax.ShapeDtypeStruct((_N_LR, _FW), jnp.float32),
    )(psums, pcnt)


def _edge_body(a_ref, w_ref, b_ref, o_ref):
    o_ref[...] = (jnp.dot(w_ref[...], a_ref[...],
                          preferred_element_type=jnp.float32) + b_ref[...])


def _edge_encode_t(edge_attr_t, wlr, blr_col):
    blk = 16000
    grid = _E_LR // blk
    return pl.pallas_call(
        _edge_body,
        grid=(grid,),
        in_specs=[
            pl.BlockSpec((3, blk), lambda i: (0, i)),
            pl.BlockSpec(wlr.shape, lambda i: (0, 0)),
            pl.BlockSpec(blr_col.shape, lambda i: (0, 0)),
        ],
        out_specs=pl.BlockSpec((_FW, blk), lambda i: (0, i)),
        out_shape=jax.ShapeDtypeStruct((_FW, _E_LR), jnp.float32),
    )(edge_attr_t, wlr, blr_col)


@jax.jit
def kernel(v, e_hr_to_lr, idxHr_to_idxLr, batch_lr, edge_index_lr, edge_attr_lr,
           W_enc, b_enc, gamma, beta, W0, b0, W1, b1, W_lr, b_lr):
    wenc = W_enc[:, 0]
    w0g = gamma[:, None] * W0.T
    w0g_e = w0g[:_FW]
    w0g_v = w0g[_FW:]
    m1 = jnp.concatenate(
        [w0g_v, jnp.ones((_D_IN, 1), jnp.float32),
         jnp.zeros((_D_IN, _D_IN - _FW - 1), jnp.float32)], axis=1)
    m2 = jnp.ones((_D_IN, 8), jnp.float32)
    r0 = wenc[None, :] @ w0g_e
    r1e = b_enc[None, :] @ w0g_e
    s0 = jnp.sum(w0g, axis=0, keepdims=True)
    c0 = beta[None, :] @ W0.T + b0[None, :]
    cs = jnp.stack(
        [jnp.sum(wenc), jnp.sum(b_enc), jnp.sum(wenc * wenc),
         jnp.sum(wenc * b_enc), jnp.sum(b_enc * b_enc),
         jnp.zeros(()), jnp.zeros(()), jnp.zeros(())])[None, :]
    w1t = W1.T
    b1_row = b1.reshape(1, _FW)

    h = _compute_h(v, e_hr_to_lr, m1, m2, r0, r1e, s0, c0, cs, w1t, b1_row)

    z2d = jnp.zeros((_NLR_PAD, _FW), jnp.float32)
    zc = jnp.zeros((_NLR_PAD, 8), jnp.float32)
    ones = jnp.ones((_CHUNK, 8), jnp.float32)
    flat_sums, flat_cnt = _sc_scatter()(h, idxHr_to_idxLr, z2d, zc, ones)
    psums = flat_sums.reshape(2, _NLR_PAD, _FW)
    pcnt = flat_cnt.reshape(2, _NLR_PAD, 8)

    v_out = _combine(psums, pcnt)
    e_out = _edge_encode_t(edge_attr_lr.T, W_lr, b_lr.reshape(_FW, 1)).T
    return (v_out, e_out)
